# Initial kernel scaffold; baseline (speedup 1.0000x reference)
import jax
import jax.numpy as jnp
from jax.experimental import pallas as pl


def kernel(x_o, x_a, W_o1, b_o1, W_s1, b_s1, W_o2, b_o2, W_s2, b_s2, disc_W, disc_b, dec1_W, dec1_b, dec2_W, dec2_b, edge_index, edge_index2, idx):
    raise NotImplementedError("write your pallas kernel here")


# speedup vs baseline: 14.5218x; 14.5218x over previous
import functools

import jax
import jax.numpy as jnp
from jax import lax
from jax.experimental import pallas as pl
from jax.experimental.pallas import tpu as pltpu
from jax.experimental.pallas import tpu_sc as plsc

N = 10000
E = 320000
FEAT = 128
H1 = 128
H2 = 64
DEC1 = 128
P = 4096

NC = 2
NS = 16
CHUNK = 128
CH = 157
EPW = CH * CHUNK
E_PAD = NS * EPW
NROW = 10008
RA = 632
RB = N - 15 * RA
B = 2000
NB = (2 * N) // B
NBH = N // B

_f32 = jnp.float32
_i32 = jnp.int32


def _sds(shape, dtype):
    return jax.ShapeDtypeStruct(shape, dtype)


def _mesh():
    return plsc.VectorSubcoreMesh(
        core_axis_name="c", subcore_axis_name="s", num_cores=NC, num_subcores=NS
    )


def _rows_split(s, fn):
    @pl.when(s < NS - 1)
    def _():
        fn(s * RA, RA)

    @pl.when(s == NS - 1)
    def _():
        fn((NS - 1) * RA, RB)


def _deg_body(dst_hbm, ones_hbm, zeros_hbm, out_hbm, idxc_v, ones_v, sh):
    c = lax.axis_index("c")
    s = lax.axis_index("s")
    wid = c * NS + s
    pltpu.sync_copy(ones_hbm, ones_v)
    _zero_rows(zeros_hbm, sh, s)
    plsc.subcore_barrier()

    def chunk(j, carry):
        pltpu.sync_copy(dst_hbm.at[pl.ds(wid * EPW + j * CHUNK, CHUNK)], idxc_v)
        pltpu.sync_copy(ones_v, sh.at[idxc_v], add=True)
        return carry

    lax.fori_loop(0, CH, chunk, 0)
    plsc.subcore_barrier()
    _rows_split(s, lambda off, nr: pltpu.sync_copy(
        sh.at[pl.ds(off, nr)], out_hbm.at[pl.ds(c * N + off, nr)]))


@functools.cache
def _deg_kernel():
    return pl.kernel(
        _deg_body,
        out_type=_sds((NC * N, 128), _f32),
        mesh=_mesh(),
        scratch_types=[
            pltpu.VMEM((CHUNK,), _i32),
            pltpu.VMEM((CHUNK, 128), _f32),
            pltpu.VMEM_SHARED((NROW, 128), _f32),
        ],
    )


def _zero_rows(zeros_hbm, sh, s):
    @pl.when(s < NS - 1)
    def _():
        pltpu.sync_copy(zeros_hbm.at[pl.ds(0, RA)], sh.at[pl.ds(s * RA, RA)])

    @pl.when(s == NS - 1)
    def _():
        pltpu.sync_copy(
            zeros_hbm.at[pl.ds(0, NROW - (NS - 1) * RA)],
            sh.at[pl.ds((NS - 1) * RA, NROW - (NS - 1) * RA)],
        )


@functools.cache
def _make_mp(D):
    def body(tab_o, tab_s, src_o, dst_o, src_s, dst_s, out_o, out_s,
             srcb, idxc_v, rows_v, sh):
        c = lax.axis_index("c")
        s = lax.axis_index("s")
        wid = c * NS + s
        for tab, srch, dsth, outh in (
            (tab_o, src_o, dst_o, out_o),
            (tab_s, src_s, dst_s, out_s),
        ):
            def init(off, nr, tab=tab):
                pltpu.sync_copy(
                    tab.at[pl.ds(c * N + off, nr)], sh.at[pl.ds(off, nr)]
                )

            _rows_split(s, init)
            pltpu.sync_copy(srch.at[pl.ds(wid * EPW, EPW)], srcb)
            plsc.subcore_barrier()

            def chunk(j, carry, tab=tab, dsth=dsth):
                pltpu.sync_copy(dsth.at[pl.ds(s * EPW + j * CHUNK, CHUNK)], idxc_v)
                pltpu.sync_copy(tab.at[srcb.at[pl.ds(j * CHUNK, CHUNK)]], rows_v)
                pltpu.sync_copy(rows_v, sh.at[idxc_v], add=True)
                return carry

            lax.fori_loop(0, CH, chunk, 0)
            plsc.subcore_barrier()

            def dump(off, nr, outh=outh):
                pltpu.sync_copy(
                    sh.at[pl.ds(off, nr)], outh.at[pl.ds(c * N + off, nr)]
                )

            _rows_split(s, dump)
            plsc.subcore_barrier()

    return pl.kernel(
        body,
        out_type=[_sds((NC * N, D), _f32)] * 2,
        mesh=_mesh(),
        scratch_types=[
            pltpu.VMEM((EPW,), _i32),
            pltpu.VMEM((CHUNK,), _i32),
            pltpu.VMEM((CHUNK, D), _f32),
            pltpu.VMEM_SHARED((NROW, D), _f32),
        ],
    )


@functools.cache
def _mp2_kernel():
    def body(tab, srch, dsth, outh, srcb, idxc_v, rows_v, sh):
        c = lax.axis_index("c")
        s = lax.axis_index("s")
        wid = c * NS + s

        def init(off, nr):
            pltpu.sync_copy(tab.at[pl.ds(c * N + off, nr)], sh.at[pl.ds(off, nr)])

        _rows_split(s, init)
        pltpu.sync_copy(srch.at[pl.ds(wid * EPW, EPW)], srcb)
        plsc.subcore_barrier()

        def chunk(j, carry):
            pltpu.sync_copy(dsth.at[pl.ds(wid * EPW + j * CHUNK, CHUNK)], idxc_v)
            pltpu.sync_copy(tab.at[srcb.at[pl.ds(j * CHUNK, CHUNK)]], rows_v)
            pltpu.sync_copy(rows_v, sh.at[idxc_v], add=True)
            return carry

        lax.fori_loop(0, CH, chunk, 0)
        plsc.subcore_barrier()

        def dump(off, nr):
            pltpu.sync_copy(sh.at[pl.ds(off, nr)], outh.at[pl.ds(c * N + off, nr)])

        _rows_split(s, dump)

    return pl.kernel(
        body,
        out_type=_sds((NC * N, 2 * H2), _f32),
        mesh=_mesh(),
        scratch_types=[
            pltpu.VMEM((EPW,), _i32),
            pltpu.VMEM((CHUNK,), _i32),
            pltpu.VMEM((CHUNK, 2 * H2), _f32),
            pltpu.VMEM_SHARED((NROW, 2 * H2), _f32),
        ],
    )


PAIR_CH = (2 * P) // (NC * NS * CHUNK)
PPW = PAIR_CH * CHUNK


def _pair_body(x2_hbm, idx_hbm, out_hbm, idx_v, rows_v):
    c = lax.axis_index("c")
    s = lax.axis_index("s")
    wid = c * NS + s
    pltpu.sync_copy(idx_hbm.at[pl.ds(wid * PPW, PPW)], idx_v)
    for j in range(PAIR_CH):
        pltpu.sync_copy(x2_hbm.at[idx_v.at[pl.ds(j * CHUNK, CHUNK)]], rows_v)
        pltpu.sync_copy(
            rows_v, out_hbm.at[pl.ds(wid * PPW + j * CHUNK, CHUNK)]
        )


@functools.cache
def _pair_kernel():
    return pl.kernel(
        _pair_body,
        out_type=_sds((2 * P, 2 * H2), _f32),
        mesh=_mesh(),
        scratch_types=[
            pltpu.VMEM((PPW,), _i32),
            pltpu.VMEM((CHUNK, 2 * H2), _f32),
        ],
    )


def _dinv(ref):
    return lax.rsqrt(1.0 + ref[...][:, :1])


def _tc1_body(x_ref, w_ref, dgo_ref, dgs_ref, to_ref, ts_ref):
    h = jnp.dot(x_ref[...], w_ref[...], preferred_element_type=_f32)
    to_ref[...] = h[:, :H1] * _dinv(dgo_ref)
    ts_ref[...] = h[:, H1:] * _dinv(dgs_ref)


def _tc2_body(so0_ref, so1_ref, ss0_ref, ss1_ref, dgo_ref, dgs_ref, b1_ref,
              w2_ref, to_ref, ts_ref):
    dio = _dinv(dgo_ref)
    dis = _dinv(dgs_ref)

    def h2(so_ref, ss_ref):
        x1o = jnp.maximum(so_ref[...] * dio + b1_ref[...][:, :H1], 0.0)
        x1s = jnp.maximum(ss_ref[...] * dis + b1_ref[...][:, H1:], 0.0)
        return jnp.dot(
            jnp.concatenate([x1o, x1s], axis=1), w2_ref[...],
            preferred_element_type=_f32,
        )

    h2e0 = h2(so0_ref, ss0_ref)
    h2e1 = h2(so1_ref, ss1_ref)
    to_ref[...] = jnp.concatenate([h2e0[:, :H2] * dio, h2e1[:, :H2] * dio], axis=1)
    ts_ref[...] = jnp.concatenate([h2e0[:, H2:] * dis, h2e1[:, H2:] * dis], axis=1)


def _tc3_body(a1_ref, a2_ref, dgo_ref, dgs_ref, b2_ref,
              x20_ref, x21_ref, ps0_ref, ps1_ref):
    dio = _dinv(dgo_ref)
    dis = _dinv(dgs_ref)
    a1 = a1_ref[...]
    a2 = a2_ref[...]
    x2e0 = jnp.concatenate(
        [a1[:, :H2] * dio + b2_ref[...][:, :H2],
         a2[:, :H2] * dis + b2_ref[...][:, H2:]], axis=1)
    x2e1 = jnp.concatenate(
        [a1[:, H2:] * dio + b2_ref[...][:, :H2],
         a2[:, H2:] * dis + b2_ref[...][:, H2:]], axis=1)
    x20_ref[...] = x2e0
    x21_ref[...] = x2e1
    ps0_ref[...] = jnp.sum(x2e0, axis=0, keepdims=True)[None]
    ps1_ref[...] = jnp.sum(x2e1, axis=0, keepdims=True)[None]


def _tc4a_body(x20_ref, x21_ref, ps0_ref, ps1_ref, dw_ref, db_ref,
               so0_ref, so1_ref, sa0_ref, sa1_ref):
    nbh = NB // 2
    ho = jax.nn.sigmoid(jnp.sum(ps0_ref[...][:nbh, 0], axis=0, keepdims=True) / N)
    ha = jax.nn.sigmoid(jnp.sum(ps1_ref[...][:nbh, 0], axis=0, keepdims=True) / N)
    g0 = jnp.dot(x20_ref[...], dw_ref[...], preferred_element_type=_f32)
    g1 = jnp.dot(x21_ref[...], dw_ref[...], preferred_element_type=_f32)
    db = db_ref[0, 0]
    so0_ref[...] = jnp.sum(g0 * ho, axis=1, keepdims=True) + db
    so1_ref[...] = jnp.sum(g1 * ho, axis=1, keepdims=True) + db
    sa0_ref[...] = jnp.sum(g0 * ha, axis=1, keepdims=True) + db
    sa1_ref[...] = jnp.sum(g1 * ha, axis=1, keepdims=True) + db


def _tc4b_body(e1_ref, e2_ref, w1_ref, b1_ref, w2_ref, b2_ref, out_ref):
    e1 = e1_ref[...]
    e2 = e2_ref[...]
    f = jnp.concatenate([e1 + e2, e1 * e2, e1, e2], axis=1)
    hr = jnp.maximum(
        jnp.dot(f, w1_ref[...], preferred_element_type=_f32) + b1_ref[...], 0.0
    )
    out_ref[...] = (
        jnp.dot(hr, w2_ref[...], preferred_element_type=_f32) + b2_ref[0, 0]
    )


def _blk_e(i):
    return (i, 0)


def _blk_r(i):
    return (i, 0)


def _blk_r2(i):
    return (NBH + i, 0)


def _blk_dego(i):
    return (i % NBH, 0)


def _blk_degs(i):
    return (NBH + i % NBH, 0)


def _blk0(i):
    return (0, 0)


def kernel(x_o, x_a, W_o1, b_o1, W_s1, b_s1, W_o2, b_o2, W_s2, b_s2,
           disc_W, disc_b, dec1_W, dec1_b, dec2_W, dec2_b,
           edge_index, edge_index2, idx):
    def prep(src, dst):
        srcp = jnp.concatenate([src, jnp.zeros((E_PAD - E,), _i32)])
        dstp = jnp.concatenate([dst, jnp.full((E_PAD - E,), N, _i32)])
        SRC = jnp.concatenate([srcp, srcp + N])
        return SRC, dstp

    SRC1, DST1 = prep(edge_index[0], edge_index[1])
    SRC2, DST2 = prep(edge_index2[0], edge_index2[1])
    DSTD = jnp.concatenate([DST1, DST2])
    IDXP = idx.reshape(2 * P)

    ones16 = jnp.ones((CHUNK, 128), _f32)
    zeros16 = jnp.zeros((RA, 128), _f32)

    deg = _deg_kernel()(DSTD, ones16, zeros16)

    x_all = jnp.concatenate([x_o, x_a], axis=0)
    Wc1 = jnp.concatenate([W_o1, W_s1], axis=1)
    tab_o, tab_s = pl.pallas_call(
        _tc1_body,
        grid=(NB,),
        in_specs=[
            pl.BlockSpec((B, FEAT), _blk_e),
            pl.BlockSpec((FEAT, 2 * H1), _blk0),
            pl.BlockSpec((B, 128), _blk_dego),
            pl.BlockSpec((B, 128), _blk_degs),
        ],
        out_specs=[
            pl.BlockSpec((B, H1), _blk_e),
            pl.BlockSpec((B, H1), _blk_e),
        ],
        out_shape=[_sds((2 * N, H1), _f32)] * 2,
    )(x_all, Wc1, deg, deg)

    acc1_o, acc1_s = _make_mp(H1)(tab_o, tab_s, SRC1, DST1, SRC2, DST2)

    b1c = jnp.concatenate([b_o1, b_s1]).reshape(1, 2 * H1)
    Wc2 = jnp.concatenate([W_o2, W_s2], axis=1)
    tab2_o, tab2_s = pl.pallas_call(
        _tc2_body,
        grid=(NBH,),
        in_specs=[
            pl.BlockSpec((B, H1), _blk_r),
            pl.BlockSpec((B, H1), _blk_r2),
            pl.BlockSpec((B, H1), _blk_r),
            pl.BlockSpec((B, H1), _blk_r2),
            pl.BlockSpec((B, 128), _blk_r),
            pl.BlockSpec((B, 128), _blk_r2),
            pl.BlockSpec((1, 2 * H1), _blk0),
            pl.BlockSpec((2 * H1, 2 * H2), _blk0),
        ],
        out_specs=[
            pl.BlockSpec((B, 2 * H2), _blk_r),
            pl.BlockSpec((B, 2 * H2), _blk_r),
        ],
        out_shape=[_sds((N, 2 * H2), _f32)] * 2,
    )(acc1_o, acc1_o, acc1_s, acc1_s, deg, deg, b1c, Wc2)

    TAB2 = jnp.concatenate([tab2_o, tab2_s], axis=0)
    SRCL2 = jnp.concatenate([SRC1[:E_PAD], SRC2[:E_PAD] + N])
    DSTL2 = jnp.concatenate([DST1, DST2])
    acc2 = _mp2_kernel()(TAB2, SRCL2, DSTL2)

    b2c = jnp.concatenate([b_o2, b_s2]).reshape(1, 2 * H2)
    x2_e0, x2_e1, ps0, ps1 = pl.pallas_call(
        _tc3_body,
        grid=(NBH,),
        in_specs=[
            pl.BlockSpec((B, 2 * H2), _blk_r),
            pl.BlockSpec((B, 2 * H2), _blk_r2),
            pl.BlockSpec((B, 128), _blk_r),
            pl.BlockSpec((B, 128), _blk_r2),
            pl.BlockSpec((1, 2 * H2), _blk0),
        ],
        out_specs=[
            pl.BlockSpec((B, 2 * H2), _blk_r),
            pl.BlockSpec((B, 2 * H2), _blk_r),
            pl.BlockSpec((1, 1, 2 * H2), lambda i: (i, 0, 0)),
            pl.BlockSpec((1, 1, 2 * H2), lambda i: (i, 0, 0)),
        ],
        out_shape=[_sds((N, 2 * H2), _f32), _sds((N, 2 * H2), _f32),
                   _sds((8, 1, 2 * H2), _f32), _sds((8, 1, 2 * H2), _f32)],
    )(acc2, acc2, deg, deg, b2c)

    so0, so1, sa0, sa1 = pl.pallas_call(
        _tc4a_body,
        grid=(NBH,),
        in_specs=[
            pl.BlockSpec((B, 2 * H2), _blk_r),
            pl.BlockSpec((B, 2 * H2), _blk_r),
            pl.BlockSpec((8, 1, 2 * H2), lambda i: (0, 0, 0)),
            pl.BlockSpec((8, 1, 2 * H2), lambda i: (0, 0, 0)),
            pl.BlockSpec((2 * H2, 2 * H2), _blk0),
            pl.BlockSpec((1, 1), _blk0),
        ],
        out_specs=[pl.BlockSpec((B, 1), _blk_r)] * 4,
        out_shape=[_sds((N, 1), _f32)] * 4,
    )(x2_e0, x2_e1, ps0, ps1, disc_W, disc_b.reshape(1, 1))

    e12 = _pair_kernel()(x2_e0, IDXP)

    log = pl.pallas_call(
        _tc4b_body,
        grid=(1,),
        in_specs=[
            pl.BlockSpec((P, 2 * H2), _blk0),
            pl.BlockSpec((P, 2 * H2), lambda i: (1, 0)),
            pl.BlockSpec((2 * H2 * 4, DEC1), _blk0),
            pl.BlockSpec((1, DEC1), _blk0),
            pl.BlockSpec((DEC1, 1), _blk0),
            pl.BlockSpec((1, 1), _blk0),
        ],
        out_specs=pl.BlockSpec((P, 1), _blk0),
        out_shape=_sds((P, 1), _f32),
    )(e12, e12, dec1_W, dec1_b.reshape(1, DEC1), dec2_W, dec2_b.reshape(1, 1))

    ret_os = jnp.concatenate([so0, so1], axis=1)
    ret_os_a = jnp.concatenate([sa1, sa0], axis=1)
    return (log, ret_os, ret_os_a, x2_e0)
